# baseline (speedup 1.0000x reference)
import functools

import jax
import jax.numpy as jnp
from jax import lax
from jax.experimental import pallas as pl
from jax.experimental.pallas import tpu as pltpu
from jax.experimental.pallas import tpu_sc as plsc

B = 16384
D = 64
NC = 2
NS = 16
L = 16
NW = NC * NS
BPW = B // NW
CHUNK = 128
NCHUNK = BPW // CHUNK
NTILE = BPW // L
L2 = 0.0001
EPS = 1e-8


def _sc_body(u_hbm, i_hbm, j_hbm, user_hbm, item_hbm, bias_hbm,
             x_out, sq_out,
             idx_u, idx_i, idx_j, ue, ie, je, bi, bj, xrow, sq_v,
             sem):
    wid = lax.axis_index("s") * NC + lax.axis_index("c")
    base = wid * BPW
    row0 = wid * NCHUNK

    pltpu.sync_copy(u_hbm.at[pl.ds(row0, NCHUNK)], idx_u)
    pltpu.sync_copy(i_hbm.at[pl.ds(row0, NCHUNK)], idx_i)
    pltpu.sync_copy(j_hbm.at[pl.ds(row0, NCHUNK)], idx_j)

    copies = []
    for c in range(NCHUNK):
        r = pl.ds(c * CHUNK, CHUNK)
        copies.append(pltpu.async_copy(user_hbm.at[idx_u.at[c]], ue.at[r], sem))
        copies.append(pltpu.async_copy(item_hbm.at[idx_i.at[c]], ie.at[r], sem))
        copies.append(pltpu.async_copy(item_hbm.at[idx_j.at[c]], je.at[r], sem))
        copies.append(pltpu.async_copy(bias_hbm.at[idx_i.at[c]], bi.at[r], sem))
        copies.append(pltpu.async_copy(bias_hbm.at[idx_j.at[c]], bj.at[r], sem))
    for cp in copies:
        cp.wait()

    iota16 = lax.iota(jnp.int32, L)

    def tile_body(t, carry):
        squ, sqi, sqj = carry
        xv = jnp.zeros((L,), jnp.float32)
        for k in range(L):
            r = t * L + k
            acc = jnp.zeros((L,), jnp.float32)
            for c in range(D // L):
                s = pl.ds(c * L, L)
                uc = ue[r, s]
                ic = ie[r, s]
                jc = je[r, s]
                acc = acc + uc * (ic - jc)
                squ = squ + uc * uc
                sqi = sqi + ic * ic
                sqj = sqj + jc * jc
            for sh in (8, 4, 2, 1):
                acc = acc + jnp.take(acc, iota16 ^ sh)
            xv = jnp.where(iota16 == k, acc, xv)
        s = pl.ds(t * L, L)
        xrow[s] = xv + bi[s] - bj[s]
        return squ, sqi, sqj

    zero = jnp.zeros((L,), jnp.float32)
    squ, sqi, sqj = lax.fori_loop(0, NTILE, tile_body, (zero, zero, zero))

    sq_v[pl.ds(0, L)] = squ
    sq_v[pl.ds(L, L)] = sqi
    sq_v[pl.ds(2 * L, L)] = sqj
    pltpu.sync_copy(xrow, x_out.at[pl.ds(base, BPW)])
    pltpu.sync_copy(sq_v, sq_out.at[wid])


def _tc_body(x_ref, sq_ref, o_ref):
    x = x_ref[...]
    loss = jnp.sum(-jnp.log(jax.nn.sigmoid(x) + EPS)) / B
    su = jnp.sum(sq_ref[:, 0:L])
    si = jnp.sum(sq_ref[:, L:2 * L])
    sj = jnp.sum(sq_ref[:, 2 * L:3 * L])
    reg = L2 * (su + si + sj) / (B * D)
    o_ref[...] = jnp.full((1, 1), loss + reg, jnp.float32)


@jax.jit
def _bprmf(u2, i2, j2, user_w, item_w, bias1):
    sc = pl.kernel(
        _sc_body,
        out_type=(
            jax.ShapeDtypeStruct((B,), jnp.float32),
            jax.ShapeDtypeStruct((NW, 3 * L), jnp.float32),
        ),
        mesh=plsc.VectorSubcoreMesh(core_axis_name="c", subcore_axis_name="s"),
        compiler_params=pltpu.CompilerParams(use_tc_tiling_on_sc=False),
        scratch_types=[
            pltpu.VMEM((NCHUNK, CHUNK), jnp.int32),
            pltpu.VMEM((NCHUNK, CHUNK), jnp.int32),
            pltpu.VMEM((NCHUNK, CHUNK), jnp.int32),
            pltpu.VMEM((BPW, D), jnp.float32),
            pltpu.VMEM((BPW, D), jnp.float32),
            pltpu.VMEM((BPW, D), jnp.float32),
            pltpu.VMEM((BPW,), jnp.float32),
            pltpu.VMEM((BPW,), jnp.float32),
            pltpu.VMEM((BPW,), jnp.float32),
            pltpu.VMEM((3 * L,), jnp.float32),
            pltpu.SemaphoreType.DMA,
        ],
    )
    x, sq = sc(u2, i2, j2, user_w, item_w, bias1)

    out = pl.pallas_call(
        _tc_body,
        out_shape=jax.ShapeDtypeStruct((1, 1), jnp.float32),
    )(x.reshape(B // CHUNK, CHUNK), sq)
    return out[0, 0]


def kernel(u, i, j, user_w, item_w, item_bias_w):
    u2 = u.astype(jnp.int32).reshape(B // CHUNK, CHUNK)
    i2 = i.astype(jnp.int32).reshape(B // CHUNK, CHUNK)
    j2 = j.astype(jnp.int32).reshape(B // CHUNK, CHUNK)
    bias1 = item_bias_w.reshape(-1)
    return _bprmf(u2, i2, j2, user_w, item_w, bias1)
